# Initial kernel scaffold; baseline (speedup 1.0000x reference)
import jax
import jax.numpy as jnp
from jax.experimental import pallas as pl


def kernel(edge_index, edge_weight, We1, be1, We2, be2, Wd1, bd1, Wd2, bd2):
    raise NotImplementedError("write your pallas kernel here")


# speedup vs baseline: 12.1080x; 12.1080x over previous
import functools

import jax
import jax.numpy as jnp
from jax import lax
from jax.experimental import pallas as pl
from jax.experimental.pallas import tpu as pltpu
from jax.experimental.pallas import tpu_sc as plsc

N = 10000
E = 320000
NC = 2
NS = 16
NW = NC * NS
CHUNK = 128
EC = 79
E_PAD = NW * EC * CHUNK
N_PAD = 10240
RPT = N_PAD // NS
DEG_W = 8

_mesh = plsc.VectorSubcoreMesh(
    core_axis_name="c", subcore_axis_name="s", num_cores=NC, num_subcores=NS
)
_sc_params = pltpu.CompilerParams(use_tc_tiling_on_sc=False)


def _make_agg(D):
  @functools.partial(
      pl.kernel,
      out_type=jax.ShapeDtypeStruct((NC, N_PAD, D), jnp.float32),
      mesh=_mesh,
      scratch_types=[
          pltpu.VMEM((EC, CHUNK), jnp.int32),
          pltpu.VMEM((EC, CHUNK), jnp.int32),
          pltpu.VMEM((CHUNK, D), jnp.float32),
          pltpu.VMEM((RPT, D), jnp.float32),
          pltpu.VMEM_SHARED((N_PAD, D), jnp.float32),
      ],
      compiler_params=_sc_params,
  )
  def agg(xs_hbm, src_hbm, dst_hbm, zeros_hbm, out_hbm,
          src_v, dst_v, rows_v, zbuf_v, acc_sh):
    cid = lax.axis_index("c")
    sid = lax.axis_index("s")
    wid = sid * NC + cid
    pltpu.sync_copy(src_hbm.at[wid], src_v)
    pltpu.sync_copy(dst_hbm.at[wid], dst_v)
    pltpu.sync_copy(zeros_hbm, zbuf_v)
    pltpu.sync_copy(zbuf_v, acc_sh.at[pl.ds(sid * RPT, RPT)])
    plsc.subcore_barrier()

    def body(j, carry):
      pltpu.sync_copy(xs_hbm.at[src_v.at[j]], rows_v)
      pltpu.sync_copy(rows_v, acc_sh.at[dst_v.at[j]], add=True)
      return carry

    lax.fori_loop(0, EC, body, 0)
    plsc.subcore_barrier()
    pltpu.sync_copy(acc_sh.at[pl.ds(sid * RPT, RPT)], zbuf_v)
    pltpu.sync_copy(zbuf_v, out_hbm.at[cid, pl.ds(sid * RPT, RPT)])

  return agg


@functools.partial(
    pl.kernel,
    out_type=jax.ShapeDtypeStruct((NC, N_PAD, DEG_W), jnp.float32),
    mesh=_mesh,
    scratch_types=[
        pltpu.VMEM((EC, CHUNK), jnp.int32),
        pltpu.VMEM((CHUNK, DEG_W), jnp.float32),
        pltpu.VMEM((RPT, DEG_W), jnp.float32),
        pltpu.VMEM_SHARED((N_PAD, DEG_W), jnp.float32),
    ],
    compiler_params=_sc_params,
)
def _deg_kernel(dst_hbm, ones_hbm, zeros_hbm, out_hbm,
                dst_v, ones_v, zbuf_v, acc_sh):
  cid = lax.axis_index("c")
  sid = lax.axis_index("s")
  wid = sid * NC + cid
  pltpu.sync_copy(dst_hbm.at[wid], dst_v)
  pltpu.sync_copy(ones_hbm, ones_v)
  pltpu.sync_copy(zeros_hbm, zbuf_v)
  pltpu.sync_copy(zbuf_v, acc_sh.at[pl.ds(sid * RPT, RPT)])
  plsc.subcore_barrier()

  def body(j, carry):
    pltpu.sync_copy(ones_v, acc_sh.at[dst_v.at[j]], add=True)
    return carry

  lax.fori_loop(0, EC, body, 0)
  plsc.subcore_barrier()
  pltpu.sync_copy(acc_sh.at[pl.ds(sid * RPT, RPT)], zbuf_v)
  pltpu.sync_copy(zbuf_v, out_hbm.at[cid, pl.ds(sid * RPT, RPT)])


def _tc_first(degA, degB, x, W):
  def body(da, db, x_ref, w_ref, xs_out, isd_out):
    deg = jnp.maximum(da[:, 0:1] + db[:, 0:1], 1.0)
    isd = lax.rsqrt(deg)
    t = jnp.dot(x_ref[...], w_ref[...], preferred_element_type=jnp.float32)
    xs_out[...] = t * isd
    isd_out[...] = jnp.broadcast_to(isd, (N, DEG_W))
  return pl.pallas_call(
      body,
      out_shape=[jax.ShapeDtypeStruct((N, W.shape[1]), jnp.float32),
                 jax.ShapeDtypeStruct((N, DEG_W), jnp.float32)],
  )(degA, degB, x, W)


def _tc_mid(aggA, aggB, isd, b, W, want_h):
  def body(aa, ab, isd_ref, b_ref, w_ref, *outs):
    isd = isd_ref[:, 0:1]
    h = jnp.maximum(isd * (aa[...] + ab[...]) + b_ref[...], 0.0)
    t = jnp.dot(h, w_ref[...], preferred_element_type=jnp.float32)
    outs[0][...] = t * isd
    if want_h:
      outs[1][...] = h
  out_shape = [jax.ShapeDtypeStruct((N, W.shape[1]), jnp.float32)]
  if want_h:
    out_shape.append(jax.ShapeDtypeStruct((N, b.shape[0]), jnp.float32))
  return pl.pallas_call(body, out_shape=out_shape)(
      aggA, aggB, isd, b.reshape(1, -1), W)


def _tc_scale(aggA, aggB, isd, b):
  def body(aa, ab, isd_ref, b_ref, out):
    isd = isd_ref[:, 0:1]
    out[...] = jnp.maximum(isd * (aa[...] + ab[...]) + b_ref[...], 0.0) * isd
  return pl.pallas_call(
      body, out_shape=jax.ShapeDtypeStruct((N, aggA.shape[1]), jnp.float32)
  )(aggA, aggB, isd, b.reshape(1, -1))


def _tc_last(aggA, aggB, isd, W, b):
  def body(aa, ab, isd_ref, w_ref, b_ref, out):
    isd = isd_ref[:, 0:1]
    t = jnp.dot(isd * (aa[...] + ab[...]), w_ref[...],
                preferred_element_type=jnp.float32)
    out[...] = t + b_ref[...]
  return pl.pallas_call(
      body, out_shape=jax.ShapeDtypeStruct((N, W.shape[1]), jnp.float32)
  )(aggA, aggB, isd, W, b.reshape(1, -1))


_agg64 = _make_agg(64)
_agg32 = _make_agg(32)


def _pad_rows(x):
  return jnp.pad(x, ((0, N_PAD - N), (0, 0)))


def kernel(edge_index, edge_weight, We1, be1, We2, be2, Wd1, bd1, Wd2, bd2):
  src = edge_index[0].astype(jnp.int32)
  dst = edge_index[1].astype(jnp.int32)
  fill = jnp.full((E_PAD - E,), N, jnp.int32)
  src_p = jnp.concatenate([src, fill]).reshape(NW, EC, CHUNK)
  dst_p = jnp.concatenate([dst, fill]).reshape(NW, EC, CHUNK)
  ones_rows = jnp.ones((CHUNK, DEG_W), jnp.float32)
  zeros8 = jnp.zeros((RPT, DEG_W), jnp.float32)
  zeros32 = jnp.zeros((RPT, 32), jnp.float32)
  zeros64 = jnp.zeros((RPT, 64), jnp.float32)

  deg2 = _deg_kernel(dst_p, ones_rows, zeros8)
  xs1, isd = _tc_first(deg2[0, :N], deg2[1, :N], edge_weight, We1)

  agg1 = _agg64(_pad_rows(xs1), src_p, dst_p, zeros64)
  xs2 = _tc_mid(agg1[0, :N], agg1[1, :N], isd, be1, We2, want_h=False)[0]

  agg2 = _agg32(_pad_rows(xs2), src_p, dst_p, zeros32)
  xs3, z = _tc_mid(agg2[0, :N], agg2[1, :N], isd, be2, Wd1, want_h=True)

  agg3 = _agg64(_pad_rows(xs3), src_p, dst_p, zeros64)
  xs4 = _tc_scale(agg3[0, :N], agg3[1, :N], isd, bd1)

  agg4 = _agg64(_pad_rows(xs4), src_p, dst_p, zeros64)
  recon = _tc_last(agg4[0, :N], agg4[1, :N], isd, Wd2, bd2)

  return (recon, z)
